# baseline (speedup 1.0000x reference)
import jax
import jax.numpy as jnp
from jax.experimental import pallas as pl


def _bn(x, g, b):
    m = jnp.mean(x, axis=0)
    v = jnp.var(x, axis=0)
    return (x - m) / jnp.sqrt(v + 1e-5) * g + b


def _lrelu(x):
    return jnp.where(x >= 0, x, 0.01 * x)


def _mlp(x, lins, bns):
    for (W, b), (g, be) in zip(lins, bns):
        x = x @ W + b
        x = _bn(x, g, be)
        x = _lrelu(x)
    return x


def _gin(x, edge_index, p):
    src = edge_index[0]
    dst = edge_index[1]
    agg = jax.ops.segment_sum(x[src], dst, num_segments=x.shape[0])
    h = (1.0 + p["eps"]) * x + agg
    return _mlp(h, p["lins"], p["bns"])


def kernel(x, edge_index, batch, params):
    NC = len(params["convs"]) - 1
    G = 64
    h = _lrelu(_gin(x, edge_index, params["convs"][0]))
    hs = []
    for i in range(1, NC + 1):
        h = _lrelu(_gin(h, edge_index, params["convs"][i]))
        hs.append(h)
    h_pool = jax.ops.segment_sum(h, batch, num_segments=G)
    h_pool_ = h_pool[batch]
    h = jnp.concatenate(hs + [h_pool_], axis=1)
    W, b = params["cls1"]
    h = h @ W + b
    for (W, b) in params["cls"]:
        h = _lrelu(h @ W + b)
    W, b = params["final"]
    h = h @ W + b
    return jax.nn.sigmoid(h)


# speedup vs baseline: 5.6217x; 5.6217x over previous
import functools

import jax
import jax.numpy as jnp
from jax import lax
from jax.experimental import pallas as pl
from jax.experimental.pallas import tpu as pltpu
from jax.experimental.pallas import tpu_sc as plsc

_NCORES = 2
_NSUB = 16
_B = 125


def _seg_sum_sc(h, src2d, dst2d):
    N, D = h.shape
    CH = src2d.shape[0]
    ch_per_tile = CH // (_NCORES * _NSUB)
    rows_per_tile = (N // _NSUB) // 8 * 8
    rem_rows = N - _NSUB * rows_per_tile
    zb = 104
    zblocks = rows_per_tile // zb

    mesh = plsc.VectorSubcoreMesh(core_axis_name="core", subcore_axis_name="subcore",
                                  num_cores=_NCORES, num_subcores=_NSUB)

    @functools.partial(
        pl.kernel,
        out_type=jax.ShapeDtypeStruct((_NCORES, N, D), jnp.float32),
        mesh=mesh,
        scratch_types=[
            pltpu.VMEM((ch_per_tile, _B), jnp.int32),
            pltpu.VMEM((ch_per_tile, _B), jnp.int32),
            pltpu.VMEM((_B, D), jnp.float32),
            pltpu.VMEM_SHARED((N, D), jnp.float32),
        ],
    )
    def seg_sum(h_hbm, src_hbm, dst_hbm, out_hbm, idx_s, idx_d, rows, acc):
        c = lax.axis_index("core")
        s = lax.axis_index("subcore")
        base = s * rows_per_tile

        @pl.loop(0, zb)
        def _(i):
            @pl.loop(0, D, step=16)
            def _(k):
                rows[pl.ds(i, 1), pl.ds(k, 16)] = jnp.zeros((1, 16), jnp.float32)

        @pl.loop(0, zblocks)
        def _(m):
            pltpu.sync_copy(rows.at[pl.ds(0, zb)], acc.at[pl.ds(base + m * zb, zb)])

        @pl.when(s == _NSUB - 1)
        def _():
            pltpu.sync_copy(rows.at[pl.ds(0, rem_rows)],
                            acc.at[pl.ds(_NSUB * rows_per_tile, rem_rows)])

        plsc.subcore_barrier()

        row0 = (c * _NSUB + s) * ch_per_tile
        pltpu.sync_copy(src_hbm.at[pl.ds(row0, ch_per_tile)], idx_s)
        pltpu.sync_copy(dst_hbm.at[pl.ds(row0, ch_per_tile)], idx_d)

        @pl.loop(0, ch_per_tile)
        def _(j):
            pltpu.sync_copy(h_hbm.at[idx_s.at[j]], rows)
            pltpu.sync_copy(rows, acc.at[idx_d.at[j]], add=True)

        plsc.subcore_barrier()
        pltpu.sync_copy(acc.at[pl.ds(base, rows_per_tile)],
                        out_hbm.at[c, pl.ds(base, rows_per_tile)])

        @pl.when(s == _NSUB - 1)
        def _():
            tail = _NSUB * rows_per_tile
            pltpu.sync_copy(acc.at[pl.ds(tail, rem_rows)],
                            out_hbm.at[c, pl.ds(tail, rem_rows)])

    return seg_sum(h, src2d, dst2d)


def _mlp_body(h_ref, a0_ref, a1_ref, er_ref,
              w1_ref, b1_ref, g1_ref, e1_ref,
              w2_ref, b2_ref, g2_ref, e2_ref, o_ref):
    h0 = h_ref[...] * er_ref[...] + (a0_ref[...] + a1_ref[...])
    y = jnp.dot(h0, w1_ref[...], preferred_element_type=jnp.float32, precision=lax.Precision.HIGHEST) + b1_ref[...]
    m = jnp.mean(y, axis=0, keepdims=True)
    v = jnp.mean((y - m) ** 2, axis=0, keepdims=True)
    y = (y - m) / jnp.sqrt(v + 1e-5) * g1_ref[...] + e1_ref[...]
    y = jnp.where(y >= 0, y, 0.01 * y)
    y = jnp.dot(y, w2_ref[...], preferred_element_type=jnp.float32, precision=lax.Precision.HIGHEST) + b2_ref[...]
    m = jnp.mean(y, axis=0, keepdims=True)
    v = jnp.mean((y - m) ** 2, axis=0, keepdims=True)
    y = (y - m) / jnp.sqrt(v + 1e-5) * g2_ref[...] + e2_ref[...]
    y = jnp.where(y >= 0, y, 0.01 * y)
    o_ref[...] = jnp.where(y >= 0, y, 0.01 * y)


def _mlp_tc(h, acc, conv):
    N, D = h.shape
    (w1, b1), (w2, b2) = conv["lins"]
    (g1, e1), (g2, e2) = conv["bns"]
    er = jnp.full((1, D), 1.0, jnp.float32) + conv["eps"]
    args = (h, acc[0], acc[1], er,
            w1, b1.reshape(1, D), g1.reshape(1, D), e1.reshape(1, D),
            w2, b2.reshape(1, D), g2.reshape(1, D), e2.reshape(1, D))
    return pl.pallas_call(
        _mlp_body,
        out_shape=jax.ShapeDtypeStruct((N, D), jnp.float32),
    )(*args)


def _pool_body(G, h3_ref, bc_ref, br_ref, o_ref):
    N, D = h3_ref.shape
    ohT = (br_ref[...] == lax.broadcasted_iota(jnp.int32, (G, N), 0)
           ).astype(jnp.float32)
    pool = jnp.dot(ohT, h3_ref[...], preferred_element_type=jnp.float32,
                   precision=lax.Precision.HIGHEST)
    oh = (bc_ref[...] == lax.broadcasted_iota(jnp.int32, (N, G), 1)
          ).astype(jnp.float32)
    o_ref[...] = jnp.dot(oh, pool, preferred_element_type=jnp.float32,
                         precision=lax.Precision.HIGHEST)


def _head_body(h1_ref, h2_ref, h3_ref, p_ref,
               wc_ref, cb_ref, wa_ref, ab_ref, wb_ref, bb_ref,
               wf_ref, fb_ref, o_ref):
    D = h1_ref.shape[1]
    hp = lax.Precision.HIGHEST
    y = (jnp.dot(h1_ref[...], wc_ref[0:D, :], preferred_element_type=jnp.float32, precision=hp)
         + jnp.dot(h2_ref[...], wc_ref[D:2 * D, :], preferred_element_type=jnp.float32, precision=hp)
         + jnp.dot(h3_ref[...], wc_ref[2 * D:3 * D, :], preferred_element_type=jnp.float32, precision=hp)
         + jnp.dot(p_ref[...], wc_ref[3 * D:4 * D, :], preferred_element_type=jnp.float32, precision=hp)
         + cb_ref[...])
    y = jnp.dot(y, wa_ref[...], preferred_element_type=jnp.float32, precision=hp) + ab_ref[...]
    y = jnp.where(y >= 0, y, 0.01 * y)
    y = jnp.dot(y, wb_ref[...], preferred_element_type=jnp.float32, precision=hp) + bb_ref[...]
    y = jnp.where(y >= 0, y, 0.01 * y)
    y = jnp.dot(y, wf_ref[...], preferred_element_type=jnp.float32, precision=hp) + fb_ref[...]
    o_ref[...] = jax.nn.sigmoid(y)


def _cls_tc(h1, h2, h3, batch, params):
    N, D = h1.shape
    wc, cb = params["cls1"]
    (wa, ab), (wb, bb) = params["cls"]
    wf, fb = params["final"]
    HID = wc.shape[1]
    G = 64
    bc = batch.reshape(N, 1)
    br = batch.reshape(1, N)
    pool_ = pl.pallas_call(
        functools.partial(_pool_body, G),
        out_shape=jax.ShapeDtypeStruct((N, D), jnp.float32),
    )(h3, bc, br)
    NB = 10
    BR = N // NB
    full = lambda shape: pl.BlockSpec(shape, lambda i: (0, 0))
    blk = lambda cols: pl.BlockSpec((BR, cols), lambda i: (i, 0))
    return pl.pallas_call(
        _head_body,
        grid=(NB,),
        in_specs=[blk(D), blk(D), blk(D), blk(D),
                  full((4 * D, HID)), full((1, HID)), full((HID, HID)),
                  full((1, HID)), full((HID, HID)), full((1, HID)),
                  full((HID, 1)), full((1, 1))],
        out_specs=blk(1),
        out_shape=jax.ShapeDtypeStruct((N, 1), jnp.float32),
    )(h1, h2, h3, pool_,
      wc, cb.reshape(1, HID), wa, ab.reshape(1, HID), wb, bb.reshape(1, HID),
      wf, fb.reshape(1, 1))


def kernel(x, edge_index, batch, params):
    N, D = x.shape
    E = edge_index.shape[1]
    src2d = edge_index[0].reshape(E // _B, _B)
    dst2d = edge_index[1].reshape(E // _B, _B)
    h = x
    hs = []
    for i, conv in enumerate(params["convs"]):
        acc = _seg_sum_sc(h, src2d, dst2d)
        h = _mlp_tc(h, acc, conv)
        if i >= 1:
            hs.append(h)
    return _cls_tc(hs[0], hs[1], hs[2], batch, params)
